# Initial kernel scaffold; baseline (speedup 1.0000x reference)
import jax
import jax.numpy as jnp
from jax.experimental import pallas as pl


def kernel(x, edge_index, batch, cheb_w0, cheb_w1, cheb_b, bn_gamma, bn_beta, fc1_w, fc1_b, bnff_gamma, bnff_beta, fc2_w, fc2_b):
    raise NotImplementedError("write your pallas kernel here")


# speedup vs baseline: 2.3887x; 2.3887x over previous
import functools

import jax
import jax.numpy as jnp
from jax import lax
from jax.experimental import pallas as pl
from jax.experimental.pallas import tpu as pltpu
from jax.experimental.pallas import tpu_sc as plsc

N = 10000
E = 320000
D = 128
H = 256
O = 64
G = 100
L = 5
EPS = 1e-5

NC = 2
NS = 16
NW = NC * NS
EW = E // NW
EC = E // NC
CH = 80
SEG = 640
NSEG = NS
NPAD = SEG * NSEG
CAP = EW + NSEG * CH
AROWS = SEG + 8

_mesh = plsc.VectorSubcoreMesh(core_axis_name="c", subcore_axis_name="s")
_f32 = jnp.float32
_i32 = jnp.int32


def _div80(x):
    return (x * 52429) >> 22


@functools.partial(
    pl.kernel,
    out_type=[
        jax.ShapeDtypeStruct((NW * CAP,), _i32),
        jax.ShapeDtypeStruct((NW * CAP,), _i32),
        jax.ShapeDtypeStruct((NW * 128,), _i32),
        jax.ShapeDtypeStruct((NW * NPAD,), _i32),
    ],
    mesh=_mesh,
    compiler_params=pltpu.CompilerParams(needs_layout_passes=False),
    scratch_types=[
        pltpu.VMEM((EW + 16,), _i32),
        pltpu.VMEM((EW + 16,), _i32),
        pltpu.VMEM((CAP,), _i32),
        pltpu.VMEM((CAP,), _i32),
        pltpu.VMEM((NPAD + 16,), _i32),
        pltpu.VMEM((128,), _i32),
        pltpu.SMEM((64,), _i32),
    ],
)
def _sc_bucket(row_hbm, col_hbm, zeros_hbm, rows_out, cols_out, meta_out,
               deg_out, rowb_v, colb_v, rout_v, cout_v, hist_v, meta_v, cnt_s):
    c = lax.axis_index("c")
    s = lax.axis_index("s")
    w = c * NS + s
    base = w * EW
    lanes = lax.iota(_i32, 16)
    lane0 = lanes == 0
    ones16 = jnp.full((16,), 1, _i32)
    pltpu.sync_copy(zeros_hbm, hist_v.at[pl.ds(0, NPAD)])
    pltpu.sync_copy(row_hbm.at[pl.ds(base, EW)], rowb_v.at[pl.ds(0, EW)])
    pltpu.sync_copy(col_hbm.at[pl.ds(base, EW)], colb_v.at[pl.ds(0, EW)])

    for b in range(NSEG):
        cnt_s[b] = 0

    def hist_body(gi, carry):
        gb = pl.multiple_of(gi * 16, 16)
        rv = rowb_v[pl.ds(gb, 16)]
        cv = colb_v[pl.ds(gb, 16)]
        for l in range(16):
            plsc.addupdate_scatter(hist_v, [jnp.full((16,), rv[l], _i32)],
                                   ones16, mask=lane0)
            bb = ((cv[l] >> 7) * 205) >> 10
            cnt_s[bb] = cnt_s[bb] + 1
        return carry

    lax.fori_loop(0, EW // 16, hist_body, 0)

    off = 0
    for b in range(NSEG):
        cnt_s[16 + b] = off
        cnt_s[32 + b] = off
        pc = _div80(cnt_s[b] + (CH - 1)) * CH

        def pad_body(k, carry, _b=b):
            kv = jnp.full((16,), k, _i32)
            plsc.store_scatter(rout_v, [kv], jnp.zeros((16,), _i32), mask=lane0)
            plsc.store_scatter(cout_v, [kv],
                               jnp.full((16,), (_b + 1) * SEG, _i32),
                               mask=lane0)
            return carry

        lax.fori_loop(off + cnt_s[b], off + pc, pad_body, 0)
        off = off + pc

    def scat_body(gi, carry):
        gb = pl.multiple_of(gi * 16, 16)
        rv = rowb_v[pl.ds(gb, 16)]
        cv = colb_v[pl.ds(gb, 16)]
        for l in range(16):
            bb = ((cv[l] >> 7) * 205) >> 10
            p = cnt_s[32 + bb]
            pv = jnp.full((16,), p, _i32)
            plsc.store_scatter(rout_v, [pv], jnp.full((16,), rv[l], _i32),
                               mask=lane0)
            plsc.store_scatter(cout_v, [pv], jnp.full((16,), cv[l], _i32),
                               mask=lane0)
            cnt_s[32 + bb] = p + 1
        return carry

    lax.fori_loop(0, EW // 16, scat_body, 0)

    for b in range(NSEG):
        plsc.store_scatter(meta_v, [jnp.full((16,), b * 8, _i32)],
                           jnp.full((16,), cnt_s[16 + b], _i32), mask=lane0)
        plsc.store_scatter(meta_v, [jnp.full((16,), b * 8 + 1, _i32)],
                           jnp.full((16,), _div80(cnt_s[b] + (CH - 1)) * CH,
                                    _i32), mask=lane0)

    pltpu.sync_copy(rout_v, rows_out.at[pl.ds(w * CAP, CAP)])
    pltpu.sync_copy(cout_v, cols_out.at[pl.ds(w * CAP, CAP)])
    pltpu.sync_copy(meta_v, meta_out.at[pl.ds(w * 128, 128)])
    pltpu.sync_copy(hist_v.at[pl.ds(0, NPAD)], deg_out.at[pl.ds(w * NPAD, NPAD)])


@functools.partial(
    pl.kernel,
    out_type=jax.ShapeDtypeStruct((NW * SEG * D,), _f32),
    mesh=_mesh,
    compiler_params=pltpu.CompilerParams(needs_layout_passes=False),
    scratch_types=[
        pltpu.VMEM((CH,), _i32),
        pltpu.VMEM((CH,), _i32),
        pltpu.VMEM((CH, D), _f32),
        pltpu.VMEM((AROWS * D,), _f32),
        pltpu.VMEM((NS * 128 + 16,), _i32),
        pltpu.SemaphoreType.DMA,
    ],
)
def _sc_scatter(rows_hbm, cols_hbm, meta_hbm, g_hbm, zeros_hbm, out_hbm,
                ridx_v, cidx_v, msg_v, accum_v, meta_v, sem):
    c = lax.axis_index("c")
    b = lax.axis_index("s")
    pltpu.sync_copy(zeros_hbm, accum_v)
    pltpu.sync_copy(meta_hbm.at[pl.ds(c * NS * 128, NS * 128)],
                    meta_v.at[pl.ds(0, NS * 128)])
    nbase = b * SEG

    def tile_body(t, carry):
        mv = meta_v[pl.ds(pl.multiple_of(t * 128 + b * 8, 8), 16)]
        off_t = mv[0]
        trip = _div80(mv[1])
        wbase = (c * NS + t) * CAP + off_t

        def chunk_body(j, cy):
            p = pl.multiple_of(wbase + j * CH, 8)
            pltpu.sync_copy(rows_hbm.at[pl.ds(p, CH)], ridx_v)
            pltpu.sync_copy(cols_hbm.at[pl.ds(p, CH)], cidx_v)
            pltpu.async_copy(g_hbm.at[ridx_v], msg_v, sem).wait()
            for g16 in range(CH // 16):
                cv = cidx_v[pl.ds(g16 * 16, 16)]
                for l in range(16):
                    e = g16 * 16 + l
                    lb = pl.multiple_of((cv[l] - nbase) * D, 16)
                    for k in range(D // 16):
                        sl = pl.ds(lb + k * 16, 16)
                        accum_v[sl] = accum_v[sl] + msg_v[e, pl.ds(k * 16, 16)]
            return cy

        lax.fori_loop(0, trip, chunk_body, 0)
        return carry

    lax.fori_loop(0, NS, tile_body, 0)

    pltpu.sync_copy(accum_v.at[pl.ds(0, SEG * D)],
                    out_hbm.at[pl.ds((c * NS + b) * SEG * D, SEG * D)])


NB = 5
BR = N // NB


def _prep_body(degp_ref, x_ref, dis_ref, g_ref):
    deg = jnp.sum(degp_ref[...], axis=1, keepdims=True)
    dis = jnp.where(deg > 0.0, lax.rsqrt(jnp.maximum(deg, 1e-12)), 0.0)
    dis_ref[...] = dis
    g_ref[...] = x_ref[...] * dis


_tc_prep = pl.pallas_call(
    _prep_body,
    grid=(NB,),
    in_specs=[
        pl.BlockSpec((BR, NW), lambda i: (i, 0)),
        pl.BlockSpec((BR, D), lambda i: (i, 0)),
    ],
    out_specs=[
        pl.BlockSpec((BR, 1), lambda i: (i, 0)),
        pl.BlockSpec((BR, D), lambda i: (i, 0)),
    ],
    out_shape=[
        jax.ShapeDtypeStruct((N, 1), _f32),
        jax.ShapeDtypeStruct((N, D), _f32),
    ],
)


def _layer_body(h_ref, t1_ref, dis_ref, w0_ref, w1_ref, b_ref, gam_ref, bet_ref,
                ho_ref, go_ref, acc_ref):
    p = pl.program_id(0)
    i = pl.program_id(1)
    dis = dis_ref[...]
    tx1 = -(dis * (t1_ref[0] + t1_ref[1]))
    u = (lax.dot_general(h_ref[...], w0_ref[...], (((1,), (1,)), ((), ())),
                         preferred_element_type=_f32)
         + lax.dot_general(tx1, w1_ref[...], (((1,), (1,)), ((), ())),
                           preferred_element_type=_f32)
         + b_ref[...])

    @pl.when(p == 0)
    def _():
        @pl.when(i == 0)
        def _():
            acc_ref[...] = jnp.zeros((8, D), _f32)
        acc_ref[0:1, :] += jnp.sum(u, axis=0, keepdims=True)
        acc_ref[1:2, :] += jnp.sum(u * u, axis=0, keepdims=True)

    @pl.when(p == 1)
    def _():
        mean = acc_ref[0:1, :] * (1.0 / N)
        var = acc_ref[1:2, :] * (1.0 / N) - mean * mean
        rstd = lax.rsqrt(var + EPS)
        hn = jnp.maximum((u - mean) * rstd * gam_ref[...] + bet_ref[...], 0.0)
        ho_ref[...] = hn
        go_ref[...] = hn * dis


_tc_layer = pl.pallas_call(
    _layer_body,
    grid=(2, NB),
    in_specs=[
        pl.BlockSpec((BR, D), lambda p, i: (i, 0)),
        pl.BlockSpec((NC, BR, D), lambda p, i: (0, i, 0)),
        pl.BlockSpec((BR, 1), lambda p, i: (i, 0)),
        pl.BlockSpec((D, D), lambda p, i: (0, 0)),
        pl.BlockSpec((D, D), lambda p, i: (0, 0)),
        pl.BlockSpec((1, D), lambda p, i: (0, 0)),
        pl.BlockSpec((1, D), lambda p, i: (0, 0)),
        pl.BlockSpec((1, D), lambda p, i: (0, 0)),
    ],
    out_specs=[
        pl.BlockSpec((BR, D), lambda p, i: (i, 0)),
        pl.BlockSpec((BR, D), lambda p, i: (i, 0)),
    ],
    out_shape=[
        jax.ShapeDtypeStruct((N, D), _f32),
        jax.ShapeDtypeStruct((N, D), _f32),
    ],
    scratch_shapes=[pltpu.VMEM((8, D), _f32)],
)


def _final_body(h_ref, bt_ref, fc1w_ref, fc1b_ref, bg_ref, bb_ref,
                fc2w_ref, fc2b_ref, out_ref, ps_ref, cnt_ref):
    i = pl.program_id(0)

    @pl.when(i == 0)
    def _():
        ps_ref[...] = jnp.zeros((104, D), _f32)
        cnt_ref[...] = jnp.zeros((104, 8), _f32)

    bt = bt_ref[...]
    M = (bt == lax.broadcasted_iota(_i32, (1, G), 1)).astype(_f32)
    ps_ref[0:G, :] += lax.dot_general(M, h_ref[...], (((0,), (0,)), ((), ())),
                                      preferred_element_type=_f32)
    cnt_ref[0:G, 0:1] += lax.dot_general(
        M, jnp.ones((BR, 1), _f32), (((0,), (0,)), ((), ())),
        preferred_element_type=_f32)

    @pl.when(i == NB - 1)
    def _():
        pooled = ps_ref[0:G, :] / jnp.maximum(cnt_ref[0:G, 0:1], 1.0)
        z = lax.dot_general(pooled, fc1w_ref[...], (((1,), (1,)), ((), ())),
                            preferred_element_type=_f32) + fc1b_ref[...]
        m = jnp.mean(z, axis=0, keepdims=True)
        v = jnp.mean((z - m) ** 2, axis=0, keepdims=True)
        z = jnp.maximum((z - m) * lax.rsqrt(v + EPS) * bg_ref[...] + bb_ref[...],
                        0.0)
        out_ref[...] = lax.dot_general(z, fc2w_ref[...], (((1,), (1,)), ((), ())),
                                       preferred_element_type=_f32) + fc2b_ref[...]


_tc_final = pl.pallas_call(
    _final_body,
    grid=(NB,),
    in_specs=[
        pl.BlockSpec((BR, D), lambda i: (i, 0)),
        pl.BlockSpec((BR, 1), lambda i: (i, 0)),
        pl.BlockSpec((H, D), lambda i: (0, 0)),
        pl.BlockSpec((1, H), lambda i: (0, 0)),
        pl.BlockSpec((1, H), lambda i: (0, 0)),
        pl.BlockSpec((1, H), lambda i: (0, 0)),
        pl.BlockSpec((O, H), lambda i: (0, 0)),
        pl.BlockSpec((1, O), lambda i: (0, 0)),
    ],
    out_specs=pl.BlockSpec((G, O), lambda i: (0, 0)),
    out_shape=jax.ShapeDtypeStruct((G, O), _f32),
    scratch_shapes=[pltpu.VMEM((104, D), _f32), pltpu.VMEM((104, 8), _f32)],
)


def kernel(x, edge_index, batch, cheb_w0, cheb_w1, cheb_b, bn_gamma, bn_beta,
           fc1_w, fc1_b, bnff_gamma, bnff_beta, fc2_w, fc2_b):
    row = edge_index[0]
    col = edge_index[1]
    zeros_i = jnp.zeros((NPAD,), _i32)
    zeros_a = jnp.zeros((AROWS * D,), _f32)

    rows_s, cols_s, meta, degh = _sc_bucket(row, col, zeros_i)
    degt = degh.reshape(NW, NPAD)[:, :N].astype(_f32).T
    dis, g = _tc_prep(degt, x)

    b2 = cheb_b.reshape(1, D)
    h = x
    for i in range(L):
        t1 = _sc_scatter(rows_s, cols_s, meta, g, zeros_a)
        t1 = t1.reshape(NC, NPAD, D)
        h, g = _tc_layer(h, t1, dis, cheb_w0, cheb_w1, b2,
                         bn_gamma[i].reshape(1, D), bn_beta[i].reshape(1, D))

    return _tc_final(h, batch.reshape(N, 1), fc1_w, fc1_b.reshape(1, H),
                     bnff_gamma.reshape(1, H), bnff_beta.reshape(1, H),
                     fc2_w, fc2_b.reshape(1, O))


# speedup vs baseline: 2.5267x; 1.0578x over previous
import functools

import jax
import jax.numpy as jnp
from jax import lax
from jax.experimental import pallas as pl
from jax.experimental.pallas import tpu as pltpu
from jax.experimental.pallas import tpu_sc as plsc

N = 10000
E = 320000
D = 128
H = 256
O = 64
G = 100
L = 5
EPS = 1e-5

NC = 2
NS = 16
NW = NC * NS
EW = E // NW
EC = E // NC
CH = 80
SEG = 640
NSEG = NS
NPAD = SEG * NSEG
CAP = EW + NSEG * CH
AROWS = SEG + 8

_mesh = plsc.VectorSubcoreMesh(core_axis_name="c", subcore_axis_name="s")
_f32 = jnp.float32
_i32 = jnp.int32


def _div80(x):
    return (x * 52429) >> 22


@functools.partial(
    pl.kernel,
    out_type=[
        jax.ShapeDtypeStruct((NW * CAP,), _i32),
        jax.ShapeDtypeStruct((NW * CAP,), _i32),
        jax.ShapeDtypeStruct((NW * 128,), _i32),
        jax.ShapeDtypeStruct((NW * NPAD,), _i32),
    ],
    mesh=_mesh,
    compiler_params=pltpu.CompilerParams(needs_layout_passes=False),
    scratch_types=[
        pltpu.VMEM((EW + 16,), _i32),
        pltpu.VMEM((EW + 16,), _i32),
        pltpu.VMEM((CAP,), _i32),
        pltpu.VMEM((CAP,), _i32),
        pltpu.VMEM((NPAD + 16,), _i32),
        pltpu.VMEM((128,), _i32),
        pltpu.SMEM((64,), _i32),
    ],
)
def _sc_bucket(row_hbm, col_hbm, zeros_hbm, rows_out, cols_out, meta_out,
               deg_out, rowb_v, colb_v, rout_v, cout_v, hist_v, meta_v, cnt_s):
    c = lax.axis_index("c")
    s = lax.axis_index("s")
    w = c * NS + s
    base = w * EW
    lanes = lax.iota(_i32, 16)
    lane0 = lanes == 0
    ones16 = jnp.full((16,), 1, _i32)
    pltpu.sync_copy(zeros_hbm, hist_v.at[pl.ds(0, NPAD)])
    pltpu.sync_copy(row_hbm.at[pl.ds(base, EW)], rowb_v.at[pl.ds(0, EW)])
    pltpu.sync_copy(col_hbm.at[pl.ds(base, EW)], colb_v.at[pl.ds(0, EW)])

    for b in range(NSEG):
        cnt_s[b] = 0

    def hist_body(gi, carry):
        gb = pl.multiple_of(gi * 16, 16)
        rv = rowb_v[pl.ds(gb, 16)]
        cv = colb_v[pl.ds(gb, 16)]
        for l in range(16):
            plsc.addupdate_scatter(hist_v, [jnp.full((16,), rv[l], _i32)],
                                   ones16, mask=lane0)
            bb = ((cv[l] >> 7) * 205) >> 10
            cnt_s[bb] = cnt_s[bb] + 1
        return carry

    lax.fori_loop(0, EW // 16, hist_body, 0)

    off = 0
    for b in range(NSEG):
        cnt_s[16 + b] = off
        cnt_s[32 + b] = off
        pc = _div80(cnt_s[b] + (CH - 1)) * CH

        def pad_body(k, carry, _b=b):
            kv = jnp.full((16,), k, _i32)
            plsc.store_scatter(rout_v, [kv], jnp.zeros((16,), _i32), mask=lane0)
            plsc.store_scatter(cout_v, [kv],
                               jnp.full((16,), (_b + 1) * SEG, _i32),
                               mask=lane0)
            return carry

        lax.fori_loop(off + cnt_s[b], off + pc, pad_body, 0)
        off = off + pc

    def scat_body(gi, carry):
        gb = pl.multiple_of(gi * 16, 16)
        rv = rowb_v[pl.ds(gb, 16)]
        cv = colb_v[pl.ds(gb, 16)]
        for l in range(16):
            bb = ((cv[l] >> 7) * 205) >> 10
            p = cnt_s[32 + bb]
            pv = jnp.full((16,), p, _i32)
            plsc.store_scatter(rout_v, [pv], jnp.full((16,), rv[l], _i32),
                               mask=lane0)
            plsc.store_scatter(cout_v, [pv], jnp.full((16,), cv[l], _i32),
                               mask=lane0)
            cnt_s[32 + bb] = p + 1
        return carry

    lax.fori_loop(0, EW // 16, scat_body, 0)

    for b in range(NSEG):
        plsc.store_scatter(meta_v, [jnp.full((16,), b * 8, _i32)],
                           jnp.full((16,), cnt_s[16 + b], _i32), mask=lane0)
        plsc.store_scatter(meta_v, [jnp.full((16,), b * 8 + 1, _i32)],
                           jnp.full((16,), _div80(cnt_s[b] + (CH - 1)) * CH,
                                    _i32), mask=lane0)

    pltpu.sync_copy(rout_v, rows_out.at[pl.ds(w * CAP, CAP)])
    pltpu.sync_copy(cout_v, cols_out.at[pl.ds(w * CAP, CAP)])
    pltpu.sync_copy(meta_v, meta_out.at[pl.ds(w * 128, 128)])
    pltpu.sync_copy(hist_v.at[pl.ds(0, NPAD)], deg_out.at[pl.ds(w * NPAD, NPAD)])


@functools.partial(
    pl.kernel,
    out_type=jax.ShapeDtypeStruct((NW * SEG * D,), _f32),
    mesh=_mesh,
    compiler_params=pltpu.CompilerParams(needs_layout_passes=False),
    scratch_types=[
        pltpu.VMEM((CH,), _i32),
        pltpu.VMEM((CH,), _i32),
        pltpu.VMEM((CH,), _i32),
        pltpu.VMEM((CH,), _i32),
        pltpu.VMEM((CH, D), _f32),
        pltpu.VMEM((CH, D), _f32),
        pltpu.VMEM((AROWS * D,), _f32),
        pltpu.VMEM((NS * 128 + 16,), _i32),
        pltpu.SemaphoreType.DMA,
        pltpu.SemaphoreType.DMA,
        pltpu.SemaphoreType.DMA,
        pltpu.SemaphoreType.DMA,
    ],
)
def _sc_scatter(rows_hbm, cols_hbm, meta_hbm, g_hbm, zeros_hbm, out_hbm,
                ridxa_v, cidxa_v, ridxb_v, cidxb_v, msga_v, msgb_v,
                accum_v, meta_v, sia, sib, sga, sgb):
    c = lax.axis_index("c")
    b = lax.axis_index("s")
    pltpu.sync_copy(zeros_hbm, accum_v)
    pltpu.sync_copy(meta_hbm.at[pl.ds(c * NS * 128, NS * 128)],
                    meta_v.at[pl.ds(0, NS * 128)])
    nbase = b * SEG

    def accumulate(cidx_v, msg_v):
        for g16 in range(CH // 16):
            cv = cidx_v[pl.ds(g16 * 16, 16)]
            for l in range(16):
                e = g16 * 16 + l
                lb = pl.multiple_of((cv[l] - nbase) * D, 16)
                for k in range(D // 16):
                    sl = pl.ds(lb + k * 16, 16)
                    accum_v[sl] = accum_v[sl] + msg_v[e, pl.ds(k * 16, 16)]

    def tile_body(t, carry):
        mv = meta_v[pl.ds(pl.multiple_of(t * 128 + b * 8, 8), 16)]
        off_t = mv[0]
        trip = _div80(mv[1])
        wbase = (c * NS + t) * CAP + off_t

        def chunk_off(j):
            return pl.multiple_of(wbase + j * CH, 8)

        @pl.when(trip > 0)
        def _():
            p = chunk_off(0)
            pltpu.sync_copy(rows_hbm.at[pl.ds(p, CH)], ridxa_v)
            pltpu.sync_copy(cols_hbm.at[pl.ds(p, CH)], cidxa_v)
            pltpu.make_async_copy(g_hbm.at[ridxa_v], msga_v, sga).start()

        @pl.when(trip > 1)
        def _():
            p = chunk_off(1)
            pltpu.make_async_copy(rows_hbm.at[pl.ds(p, CH)], ridxb_v, sib).start()
            pltpu.make_async_copy(cols_hbm.at[pl.ds(p, CH)], cidxb_v, sib).start()

        def pair_body(m, cy):
            j0 = 2 * m
            j1 = j0 + 1
            j2 = j0 + 2
            j3 = j0 + 3
            pltpu.make_async_copy(g_hbm.at[ridxa_v], msga_v, sga).wait()

            @pl.when(j1 < trip)
            def _():
                p = chunk_off(j1)
                pltpu.make_async_copy(rows_hbm.at[pl.ds(p, CH)], ridxb_v,
                                      sib).wait()
                pltpu.make_async_copy(cols_hbm.at[pl.ds(p, CH)], cidxb_v,
                                      sib).wait()
                pltpu.make_async_copy(g_hbm.at[ridxb_v], msgb_v, sgb).start()

            accumulate(cidxa_v, msga_v)

            @pl.when(j2 < trip)
            def _():
                p = chunk_off(j2)
                pltpu.make_async_copy(rows_hbm.at[pl.ds(p, CH)], ridxa_v,
                                      sia).start()
                pltpu.make_async_copy(cols_hbm.at[pl.ds(p, CH)], cidxa_v,
                                      sia).start()

            @pl.when(j1 < trip)
            def _():
                pltpu.make_async_copy(g_hbm.at[ridxb_v], msgb_v, sgb).wait()
                accumulate(cidxb_v, msgb_v)

                @pl.when(j3 < trip)
                def _():
                    p = chunk_off(j3)
                    pltpu.make_async_copy(rows_hbm.at[pl.ds(p, CH)], ridxb_v,
                                          sib).start()
                    pltpu.make_async_copy(cols_hbm.at[pl.ds(p, CH)], cidxb_v,
                                          sib).start()

            @pl.when(j2 < trip)
            def _():
                p = chunk_off(j2)
                pltpu.make_async_copy(rows_hbm.at[pl.ds(p, CH)], ridxa_v,
                                      sia).wait()
                pltpu.make_async_copy(cols_hbm.at[pl.ds(p, CH)], cidxa_v,
                                      sia).wait()
                pltpu.make_async_copy(g_hbm.at[ridxa_v], msga_v, sga).start()

            return cy

        lax.fori_loop(0, (trip + 1) >> 1, pair_body, 0)
        return carry

    lax.fori_loop(0, NS, tile_body, 0)

    pltpu.sync_copy(accum_v.at[pl.ds(0, SEG * D)],
                    out_hbm.at[pl.ds((c * NS + b) * SEG * D, SEG * D)])


NB = 5
BR = N // NB


def _prep_body(degp_ref, x_ref, dis_ref, g_ref):
    deg = jnp.sum(degp_ref[...], axis=1, keepdims=True)
    dis = jnp.where(deg > 0.0, lax.rsqrt(jnp.maximum(deg, 1e-12)), 0.0)
    dis_ref[...] = dis
    g_ref[...] = x_ref[...] * dis


_tc_prep = pl.pallas_call(
    _prep_body,
    grid=(NB,),
    in_specs=[
        pl.BlockSpec((BR, NW), lambda i: (i, 0)),
        pl.BlockSpec((BR, D), lambda i: (i, 0)),
    ],
    out_specs=[
        pl.BlockSpec((BR, 1), lambda i: (i, 0)),
        pl.BlockSpec((BR, D), lambda i: (i, 0)),
    ],
    out_shape=[
        jax.ShapeDtypeStruct((N, 1), _f32),
        jax.ShapeDtypeStruct((N, D), _f32),
    ],
)


def _layer_body(h_ref, t1_ref, dis_ref, w0_ref, w1_ref, b_ref, gam_ref, bet_ref,
                ho_ref, go_ref, acc_ref):
    p = pl.program_id(0)
    i = pl.program_id(1)
    dis = dis_ref[...]
    tx1 = -(dis * (t1_ref[0] + t1_ref[1]))
    u = (lax.dot_general(h_ref[...], w0_ref[...], (((1,), (1,)), ((), ())),
                         preferred_element_type=_f32)
         + lax.dot_general(tx1, w1_ref[...], (((1,), (1,)), ((), ())),
                           preferred_element_type=_f32)
         + b_ref[...])

    @pl.when(p == 0)
    def _():
        @pl.when(i == 0)
        def _():
            acc_ref[...] = jnp.zeros((8, D), _f32)
        acc_ref[0:1, :] += jnp.sum(u, axis=0, keepdims=True)

    @pl.when(p == 1)
    def _():
        @pl.when(i == 0)
        def _():
            acc_ref[2:3, :] = acc_ref[0:1, :] * (1.0 / N)
        d = u - acc_ref[2:3, :]
        acc_ref[1:2, :] += jnp.sum(d * d, axis=0, keepdims=True)

    @pl.when(p == 2)
    def _():
        @pl.when(i == 0)
        def _():
            acc_ref[3:4, :] = lax.rsqrt(acc_ref[1:2, :] * (1.0 / N) + EPS)
        hn = jnp.maximum((u - acc_ref[2:3, :]) * acc_ref[3:4, :]
                         * gam_ref[...] + bet_ref[...], 0.0)
        ho_ref[...] = hn
        go_ref[...] = hn * dis


_tc_layer = pl.pallas_call(
    _layer_body,
    grid=(3, NB),
    in_specs=[
        pl.BlockSpec((BR, D), lambda p, i: (i, 0)),
        pl.BlockSpec((NC, BR, D), lambda p, i: (0, i, 0)),
        pl.BlockSpec((BR, 1), lambda p, i: (i, 0)),
        pl.BlockSpec((D, D), lambda p, i: (0, 0)),
        pl.BlockSpec((D, D), lambda p, i: (0, 0)),
        pl.BlockSpec((1, D), lambda p, i: (0, 0)),
        pl.BlockSpec((1, D), lambda p, i: (0, 0)),
        pl.BlockSpec((1, D), lambda p, i: (0, 0)),
    ],
    out_specs=[
        pl.BlockSpec((BR, D), lambda p, i: (i, 0)),
        pl.BlockSpec((BR, D), lambda p, i: (i, 0)),
    ],
    out_shape=[
        jax.ShapeDtypeStruct((N, D), _f32),
        jax.ShapeDtypeStruct((N, D), _f32),
    ],
    scratch_shapes=[pltpu.VMEM((8, D), _f32)],
)


def _final_body(h_ref, bt_ref, fc1w_ref, fc1b_ref, bg_ref, bb_ref,
                fc2w_ref, fc2b_ref, out_ref, ps_ref, cnt_ref):
    i = pl.program_id(0)

    @pl.when(i == 0)
    def _():
        ps_ref[...] = jnp.zeros((104, D), _f32)
        cnt_ref[...] = jnp.zeros((104, 8), _f32)

    bt = bt_ref[...]
    M = (bt == lax.broadcasted_iota(_i32, (1, G), 1)).astype(_f32)
    ps_ref[0:G, :] += lax.dot_general(M, h_ref[...], (((0,), (0,)), ((), ())),
                                      preferred_element_type=_f32)
    cnt_ref[0:G, 0:1] += lax.dot_general(
        M, jnp.ones((BR, 1), _f32), (((0,), (0,)), ((), ())),
        preferred_element_type=_f32)

    @pl.when(i == NB - 1)
    def _():
        pooled = ps_ref[0:G, :] / jnp.maximum(cnt_ref[0:G, 0:1], 1.0)
        z = lax.dot_general(pooled, fc1w_ref[...], (((1,), (1,)), ((), ())),
                            preferred_element_type=_f32) + fc1b_ref[...]
        m = jnp.mean(z, axis=0, keepdims=True)
        v = jnp.mean((z - m) ** 2, axis=0, keepdims=True)
        z = jnp.maximum((z - m) * lax.rsqrt(v + EPS) * bg_ref[...] + bb_ref[...],
                        0.0)
        out_ref[...] = lax.dot_general(z, fc2w_ref[...], (((1,), (1,)), ((), ())),
                                       preferred_element_type=_f32) + fc2b_ref[...]


_tc_final = pl.pallas_call(
    _final_body,
    grid=(NB,),
    in_specs=[
        pl.BlockSpec((BR, D), lambda i: (i, 0)),
        pl.BlockSpec((BR, 1), lambda i: (i, 0)),
        pl.BlockSpec((H, D), lambda i: (0, 0)),
        pl.BlockSpec((1, H), lambda i: (0, 0)),
        pl.BlockSpec((1, H), lambda i: (0, 0)),
        pl.BlockSpec((1, H), lambda i: (0, 0)),
        pl.BlockSpec((O, H), lambda i: (0, 0)),
        pl.BlockSpec((1, O), lambda i: (0, 0)),
    ],
    out_specs=pl.BlockSpec((G, O), lambda i: (0, 0)),
    out_shape=jax.ShapeDtypeStruct((G, O), _f32),
    scratch_shapes=[pltpu.VMEM((104, D), _f32), pltpu.VMEM((104, 8), _f32)],
)


def kernel(x, edge_index, batch, cheb_w0, cheb_w1, cheb_b, bn_gamma, bn_beta,
           fc1_w, fc1_b, bnff_gamma, bnff_beta, fc2_w, fc2_b):
    row = edge_index[0]
    col = edge_index[1]
    zeros_i = jnp.zeros((NPAD,), _i32)
    zeros_a = jnp.zeros((AROWS * D,), _f32)

    rows_s, cols_s, meta, degh = _sc_bucket(row, col, zeros_i)
    degt = degh.reshape(NW, NPAD)[:, :N].astype(_f32).T
    dis, g = _tc_prep(degt, x)

    b2 = cheb_b.reshape(1, D)
    h = x
    for i in range(L):
        t1 = _sc_scatter(rows_s, cols_s, meta, g, zeros_a)
        t1 = t1.reshape(NC, NPAD, D)
        h, g = _tc_layer(h, t1, dis, cheb_w0, cheb_w1, b2,
                         bn_gamma[i].reshape(1, D), bn_beta[i].reshape(1, D))

    return _tc_final(h, batch.reshape(N, 1), fc1_w, fc1_b.reshape(1, H),
                     bnff_gamma.reshape(1, H), bnff_beta.reshape(1, H),
                     fc2_w, fc2_b.reshape(1, O))


# speedup vs baseline: 2.6844x; 1.0624x over previous
import functools

import jax
import jax.numpy as jnp
from jax import lax
from jax.experimental import pallas as pl
from jax.experimental.pallas import tpu as pltpu
from jax.experimental.pallas import tpu_sc as plsc

N = 10000
E = 320000
D = 128
H = 256
O = 64
G = 100
L = 5
EPS = 1e-5

NC = 2
NS = 16
NW = NC * NS
EW = E // NW
EC = E // NC
CH = 80
SEG = 640
NSEG = NS
NPAD = SEG * NSEG
CAP = EW + NSEG * CH
AROWS = SEG + 8

_mesh = plsc.VectorSubcoreMesh(core_axis_name="c", subcore_axis_name="s")
_f32 = jnp.float32
_i32 = jnp.int32


def _div80(x):
    return (x * 52429) >> 22


@functools.partial(
    pl.kernel,
    out_type=[
        jax.ShapeDtypeStruct((NW * CAP,), _i32),
        jax.ShapeDtypeStruct((NW * CAP,), _i32),
        jax.ShapeDtypeStruct((NW * 128,), _i32),
        jax.ShapeDtypeStruct((NW * NPAD,), _i32),
    ],
    mesh=_mesh,
    compiler_params=pltpu.CompilerParams(needs_layout_passes=False),
    scratch_types=[
        pltpu.VMEM((EW + 16,), _i32),
        pltpu.VMEM((EW + 16,), _i32),
        pltpu.VMEM((CAP,), _i32),
        pltpu.VMEM((CAP,), _i32),
        pltpu.VMEM((NPAD + 16,), _i32),
        pltpu.VMEM((128,), _i32),
        pltpu.SMEM((64,), _i32),
    ],
)
def _sc_bucket(row_hbm, col_hbm, zeros_hbm, rows_out, cols_out, meta_out,
               deg_out, rowb_v, colb_v, rout_v, cout_v, hist_v, meta_v, cnt_s):
    c = lax.axis_index("c")
    s = lax.axis_index("s")
    w = c * NS + s
    base = w * EW
    lanes = lax.iota(_i32, 16)
    lane0 = lanes == 0
    ones16 = jnp.full((16,), 1, _i32)
    pltpu.sync_copy(zeros_hbm, hist_v.at[pl.ds(0, NPAD)])
    pltpu.sync_copy(row_hbm.at[pl.ds(base, EW)], rowb_v.at[pl.ds(0, EW)])
    pltpu.sync_copy(col_hbm.at[pl.ds(base, EW)], colb_v.at[pl.ds(0, EW)])

    for b in range(NSEG):
        cnt_s[b] = 0

    def hist_body(gi, carry):
        gb = pl.multiple_of(gi * 16, 16)
        rv = rowb_v[pl.ds(gb, 16)]
        cv = colb_v[pl.ds(gb, 16)]
        for l in range(16):
            plsc.addupdate_scatter(hist_v, [jnp.full((16,), rv[l], _i32)],
                                   ones16, mask=lane0)
            bb = ((cv[l] >> 7) * 205) >> 10
            cnt_s[bb] = cnt_s[bb] + 1
        return carry

    lax.fori_loop(0, EW // 16, hist_body, 0)

    off = 0
    for b in range(NSEG):
        cnt_s[16 + b] = off
        cnt_s[32 + b] = off
        pc = _div80(cnt_s[b] + (CH - 1)) * CH

        def pad_body(k, carry, _b=b):
            kv = jnp.full((16,), k, _i32)
            plsc.store_scatter(rout_v, [kv], jnp.zeros((16,), _i32), mask=lane0)
            plsc.store_scatter(cout_v, [kv],
                               jnp.full((16,), (_b + 1) * SEG, _i32),
                               mask=lane0)
            return carry

        lax.fori_loop(off + cnt_s[b], off + pc, pad_body, 0)
        off = off + pc

    def scat_body(gi, carry):
        gb = pl.multiple_of(gi * 16, 16)
        rv = rowb_v[pl.ds(gb, 16)]
        cv = colb_v[pl.ds(gb, 16)]
        for l in range(16):
            bb = ((cv[l] >> 7) * 205) >> 10
            p = cnt_s[32 + bb]
            pv = jnp.full((16,), p, _i32)
            plsc.store_scatter(rout_v, [pv], jnp.full((16,), rv[l], _i32),
                               mask=lane0)
            plsc.store_scatter(cout_v, [pv], jnp.full((16,), cv[l], _i32),
                               mask=lane0)
            cnt_s[32 + bb] = p + 1
        return carry

    lax.fori_loop(0, EW // 16, scat_body, 0)

    for b in range(NSEG):
        plsc.store_scatter(meta_v, [jnp.full((16,), b * 8, _i32)],
                           jnp.full((16,), cnt_s[16 + b], _i32), mask=lane0)
        plsc.store_scatter(meta_v, [jnp.full((16,), b * 8 + 1, _i32)],
                           jnp.full((16,), _div80(cnt_s[b] + (CH - 1)) * CH,
                                    _i32), mask=lane0)

    pltpu.sync_copy(rout_v, rows_out.at[pl.ds(w * CAP, CAP)])
    pltpu.sync_copy(cout_v, cols_out.at[pl.ds(w * CAP, CAP)])
    pltpu.sync_copy(meta_v, meta_out.at[pl.ds(w * 128, 128)])
    pltpu.sync_copy(hist_v.at[pl.ds(0, NPAD)], deg_out.at[pl.ds(w * NPAD, NPAD)])


@functools.partial(
    pl.kernel,
    out_type=jax.ShapeDtypeStruct((NW * SEG * D,), _f32),
    mesh=_mesh,
    compiler_params=pltpu.CompilerParams(needs_layout_passes=False),
    scratch_types=[
        pltpu.VMEM((CH,), _i32),
        pltpu.VMEM((CH,), _i32),
        pltpu.VMEM((CH,), _i32),
        pltpu.VMEM((CH,), _i32),
        pltpu.VMEM((CH, D), _f32),
        pltpu.VMEM((CH, D), _f32),
        pltpu.VMEM((AROWS * D,), _f32),
        pltpu.VMEM((NS * 128 + 16,), _i32),
        pltpu.SemaphoreType.DMA,
        pltpu.SemaphoreType.DMA,
        pltpu.SemaphoreType.DMA,
        pltpu.SemaphoreType.DMA,
    ],
)
def _sc_scatter(rows_hbm, cols_hbm, meta_hbm, g_hbm, zeros_hbm, out_hbm,
                ridxa_v, cidxa_v, ridxb_v, cidxb_v, msga_v, msgb_v,
                accum_v, meta_v, sia, sib, sga, sgb):
    c = lax.axis_index("c")
    b = lax.axis_index("s")
    pltpu.sync_copy(zeros_hbm, accum_v)
    pltpu.sync_copy(meta_hbm.at[pl.ds(c * NS * 128, NS * 128)],
                    meta_v.at[pl.ds(0, NS * 128)])
    nbase = b * SEG

    def accumulate(cidx_v, msg_v):
        for g16 in range(CH // 16):
            cv = cidx_v[pl.ds(g16 * 16, 16)]
            for l in range(16):
                e = g16 * 16 + l
                lb = pl.multiple_of((cv[l] - nbase) * D, 16)
                for k in range(D // 16):
                    plsc.addupdate(accum_v.at[pl.ds(lb + k * 16, 16)],
                                   msg_v[e, pl.ds(k * 16, 16)])

    def tile_body(t, carry):
        mv = meta_v[pl.ds(pl.multiple_of(t * 128 + b * 8, 8), 16)]
        off_t = mv[0]
        trip = _div80(mv[1])
        wbase = (c * NS + t) * CAP + off_t

        def chunk_off(j):
            return pl.multiple_of(wbase + j * CH, 8)

        @pl.when(trip > 0)
        def _():
            p = chunk_off(0)
            pltpu.sync_copy(rows_hbm.at[pl.ds(p, CH)], ridxa_v)
            pltpu.sync_copy(cols_hbm.at[pl.ds(p, CH)], cidxa_v)
            pltpu.make_async_copy(g_hbm.at[ridxa_v], msga_v, sga).start()

        @pl.when(trip > 1)
        def _():
            p = chunk_off(1)
            pltpu.make_async_copy(rows_hbm.at[pl.ds(p, CH)], ridxb_v, sib).start()
            pltpu.make_async_copy(cols_hbm.at[pl.ds(p, CH)], cidxb_v, sib).start()

        def pair_body(m, cy):
            j0 = 2 * m
            j1 = j0 + 1
            j2 = j0 + 2
            j3 = j0 + 3
            pltpu.make_async_copy(g_hbm.at[ridxa_v], msga_v, sga).wait()

            @pl.when(j1 < trip)
            def _():
                p = chunk_off(j1)
                pltpu.make_async_copy(rows_hbm.at[pl.ds(p, CH)], ridxb_v,
                                      sib).wait()
                pltpu.make_async_copy(cols_hbm.at[pl.ds(p, CH)], cidxb_v,
                                      sib).wait()
                pltpu.make_async_copy(g_hbm.at[ridxb_v], msgb_v, sgb).start()

            accumulate(cidxa_v, msga_v)

            @pl.when(j2 < trip)
            def _():
                p = chunk_off(j2)
                pltpu.make_async_copy(rows_hbm.at[pl.ds(p, CH)], ridxa_v,
                                      sia).start()
                pltpu.make_async_copy(cols_hbm.at[pl.ds(p, CH)], cidxa_v,
                                      sia).start()

            @pl.when(j1 < trip)
            def _():
                pltpu.make_async_copy(g_hbm.at[ridxb_v], msgb_v, sgb).wait()
                accumulate(cidxb_v, msgb_v)

                @pl.when(j3 < trip)
                def _():
                    p = chunk_off(j3)
                    pltpu.make_async_copy(rows_hbm.at[pl.ds(p, CH)], ridxb_v,
                                          sib).start()
                    pltpu.make_async_copy(cols_hbm.at[pl.ds(p, CH)], cidxb_v,
                                          sib).start()

            @pl.when(j2 < trip)
            def _():
                p = chunk_off(j2)
                pltpu.make_async_copy(rows_hbm.at[pl.ds(p, CH)], ridxa_v,
                                      sia).wait()
                pltpu.make_async_copy(cols_hbm.at[pl.ds(p, CH)], cidxa_v,
                                      sia).wait()
                pltpu.make_async_copy(g_hbm.at[ridxa_v], msga_v, sga).start()

            return cy

        lax.fori_loop(0, (trip + 1) >> 1, pair_body, 0)
        return carry

    lax.fori_loop(0, NS, tile_body, 0)

    pltpu.sync_copy(accum_v.at[pl.ds(0, SEG * D)],
                    out_hbm.at[pl.ds((c * NS + b) * SEG * D, SEG * D)])


NB = 5
BR = N // NB


def _prep_body(degp_ref, x_ref, dis_ref, g_ref):
    deg = jnp.sum(degp_ref[...], axis=1, keepdims=True)
    dis = jnp.where(deg > 0.0, lax.rsqrt(jnp.maximum(deg, 1e-12)), 0.0)
    dis_ref[...] = dis
    g_ref[...] = x_ref[...] * dis


_tc_prep = pl.pallas_call(
    _prep_body,
    grid=(NB,),
    in_specs=[
        pl.BlockSpec((BR, NW), lambda i: (i, 0)),
        pl.BlockSpec((BR, D), lambda i: (i, 0)),
    ],
    out_specs=[
        pl.BlockSpec((BR, 1), lambda i: (i, 0)),
        pl.BlockSpec((BR, D), lambda i: (i, 0)),
    ],
    out_shape=[
        jax.ShapeDtypeStruct((N, 1), _f32),
        jax.ShapeDtypeStruct((N, D), _f32),
    ],
)


def _layer_body(h_ref, t1_ref, dis_ref, w0_ref, w1_ref, b_ref, gam_ref, bet_ref,
                ho_ref, go_ref, acc_ref):
    p = pl.program_id(0)
    i = pl.program_id(1)
    dis = dis_ref[...]
    tx1 = -(dis * (t1_ref[0] + t1_ref[1]))
    u = (lax.dot_general(h_ref[...], w0_ref[...], (((1,), (1,)), ((), ())),
                         preferred_element_type=_f32)
         + lax.dot_general(tx1, w1_ref[...], (((1,), (1,)), ((), ())),
                           preferred_element_type=_f32)
         + b_ref[...])

    @pl.when(p == 0)
    def _():
        @pl.when(i == 0)
        def _():
            acc_ref[...] = jnp.zeros((8, D), _f32)
        acc_ref[0:1, :] += jnp.sum(u, axis=0, keepdims=True)

    @pl.when(p == 1)
    def _():
        @pl.when(i == 0)
        def _():
            acc_ref[2:3, :] = acc_ref[0:1, :] * (1.0 / N)
        d = u - acc_ref[2:3, :]
        acc_ref[1:2, :] += jnp.sum(d * d, axis=0, keepdims=True)

    @pl.when(p == 2)
    def _():
        @pl.when(i == 0)
        def _():
            acc_ref[3:4, :] = lax.rsqrt(acc_ref[1:2, :] * (1.0 / N) + EPS)
        hn = jnp.maximum((u - acc_ref[2:3, :]) * acc_ref[3:4, :]
                         * gam_ref[...] + bet_ref[...], 0.0)
        ho_ref[...] = hn
        go_ref[...] = hn * dis


_tc_layer = pl.pallas_call(
    _layer_body,
    grid=(3, NB),
    in_specs=[
        pl.BlockSpec((BR, D), lambda p, i: (i, 0)),
        pl.BlockSpec((NC, BR, D), lambda p, i: (0, i, 0)),
        pl.BlockSpec((BR, 1), lambda p, i: (i, 0)),
        pl.BlockSpec((D, D), lambda p, i: (0, 0)),
        pl.BlockSpec((D, D), lambda p, i: (0, 0)),
        pl.BlockSpec((1, D), lambda p, i: (0, 0)),
        pl.BlockSpec((1, D), lambda p, i: (0, 0)),
        pl.BlockSpec((1, D), lambda p, i: (0, 0)),
    ],
    out_specs=[
        pl.BlockSpec((BR, D), lambda p, i: (i, 0)),
        pl.BlockSpec((BR, D), lambda p, i: (i, 0)),
    ],
    out_shape=[
        jax.ShapeDtypeStruct((N, D), _f32),
        jax.ShapeDtypeStruct((N, D), _f32),
    ],
    scratch_shapes=[pltpu.VMEM((8, D), _f32)],
)


def _final_body(h_ref, bt_ref, fc1w_ref, fc1b_ref, bg_ref, bb_ref,
                fc2w_ref, fc2b_ref, out_ref, ps_ref, cnt_ref):
    i = pl.program_id(0)

    @pl.when(i == 0)
    def _():
        ps_ref[...] = jnp.zeros((104, D), _f32)
        cnt_ref[...] = jnp.zeros((104, 8), _f32)

    bt = bt_ref[...]
    M = (bt == lax.broadcasted_iota(_i32, (1, G), 1)).astype(_f32)
    ps_ref[0:G, :] += lax.dot_general(M, h_ref[...], (((0,), (0,)), ((), ())),
                                      preferred_element_type=_f32)
    cnt_ref[0:G, 0:1] += lax.dot_general(
        M, jnp.ones((BR, 1), _f32), (((0,), (0,)), ((), ())),
        preferred_element_type=_f32)

    @pl.when(i == NB - 1)
    def _():
        pooled = ps_ref[0:G, :] / jnp.maximum(cnt_ref[0:G, 0:1], 1.0)
        z = lax.dot_general(pooled, fc1w_ref[...], (((1,), (1,)), ((), ())),
                            preferred_element_type=_f32) + fc1b_ref[...]
        m = jnp.mean(z, axis=0, keepdims=True)
        v = jnp.mean((z - m) ** 2, axis=0, keepdims=True)
        z = jnp.maximum((z - m) * lax.rsqrt(v + EPS) * bg_ref[...] + bb_ref[...],
                        0.0)
        out_ref[...] = lax.dot_general(z, fc2w_ref[...], (((1,), (1,)), ((), ())),
                                       preferred_element_type=_f32) + fc2b_ref[...]


_tc_final = pl.pallas_call(
    _final_body,
    grid=(NB,),
    in_specs=[
        pl.BlockSpec((BR, D), lambda i: (i, 0)),
        pl.BlockSpec((BR, 1), lambda i: (i, 0)),
        pl.BlockSpec((H, D), lambda i: (0, 0)),
        pl.BlockSpec((1, H), lambda i: (0, 0)),
        pl.BlockSpec((1, H), lambda i: (0, 0)),
        pl.BlockSpec((1, H), lambda i: (0, 0)),
        pl.BlockSpec((O, H), lambda i: (0, 0)),
        pl.BlockSpec((1, O), lambda i: (0, 0)),
    ],
    out_specs=pl.BlockSpec((G, O), lambda i: (0, 0)),
    out_shape=jax.ShapeDtypeStruct((G, O), _f32),
    scratch_shapes=[pltpu.VMEM((104, D), _f32), pltpu.VMEM((104, 8), _f32)],
)


def kernel(x, edge_index, batch, cheb_w0, cheb_w1, cheb_b, bn_gamma, bn_beta,
           fc1_w, fc1_b, bnff_gamma, bnff_beta, fc2_w, fc2_b):
    row = edge_index[0]
    col = edge_index[1]
    zeros_i = jnp.zeros((NPAD,), _i32)
    zeros_a = jnp.zeros((AROWS * D,), _f32)

    rows_s, cols_s, meta, degh = _sc_bucket(row, col, zeros_i)
    degt = degh.reshape(NW, NPAD)[:, :N].astype(_f32).T
    dis, g = _tc_prep(degt, x)

    b2 = cheb_b.reshape(1, D)
    h = x
    for i in range(L):
        t1 = _sc_scatter(rows_s, cols_s, meta, g, zeros_a)
        t1 = t1.reshape(NC, NPAD, D)
        h, g = _tc_layer(h, t1, dis, cheb_w0, cheb_w1, b2,
                         bn_gamma[i].reshape(1, D), bn_beta[i].reshape(1, D))

    return _tc_final(h, batch.reshape(N, 1), fc1_w, fc1_b.reshape(1, H),
                     bnff_gamma.reshape(1, H), bnff_beta.reshape(1, H),
                     fc2_w, fc2_b.reshape(1, O))


# speedup vs baseline: 2.7313x; 1.0174x over previous
import functools

import jax
import jax.numpy as jnp
from jax import lax
from jax.experimental import pallas as pl
from jax.experimental.pallas import tpu as pltpu
from jax.experimental.pallas import tpu_sc as plsc

N = 10000
E = 320000
D = 128
H = 256
O = 64
G = 100
L = 5
EPS = 1e-5

NC = 2
NS = 16
NW = NC * NS
EW = E // NW
EC = E // NC
CH = 80
SEG = 640
NSEG = NS
NPAD = SEG * NSEG
CAP = EW + NSEG * CH
AROWS = SEG + 8

_mesh = plsc.VectorSubcoreMesh(core_axis_name="c", subcore_axis_name="s")
_f32 = jnp.float32
_i32 = jnp.int32


def _div80(x):
    return (x * 52429) >> 22


@functools.partial(
    pl.kernel,
    out_type=[
        jax.ShapeDtypeStruct((NW * CAP,), _i32),
        jax.ShapeDtypeStruct((NW * CAP,), _i32),
        jax.ShapeDtypeStruct((NW * 128,), _i32),
        jax.ShapeDtypeStruct((NW * NPAD,), _i32),
    ],
    mesh=_mesh,
    compiler_params=pltpu.CompilerParams(needs_layout_passes=False),
    scratch_types=[
        pltpu.VMEM((EW + 16,), _i32),
        pltpu.VMEM((EW + 16,), _i32),
        pltpu.VMEM((CAP,), _i32),
        pltpu.VMEM((CAP,), _i32),
        pltpu.VMEM((NPAD + 16,), _i32),
        pltpu.VMEM((128,), _i32),
        pltpu.SMEM((64,), _i32),
    ],
)
def _sc_bucket(row_hbm, col_hbm, zeros_hbm, rows_out, cols_out, meta_out,
               deg_out, rowb_v, colb_v, rout_v, cout_v, hist_v, meta_v, cnt_s):
    c = lax.axis_index("c")
    s = lax.axis_index("s")
    w = c * NS + s
    base = w * EW
    lanes = lax.iota(_i32, 16)
    lane0 = lanes == 0
    ones16 = jnp.full((16,), 1, _i32)
    pltpu.sync_copy(zeros_hbm, hist_v.at[pl.ds(0, NPAD)])
    pltpu.sync_copy(row_hbm.at[pl.ds(base, EW)], rowb_v.at[pl.ds(0, EW)])
    pltpu.sync_copy(col_hbm.at[pl.ds(base, EW)], colb_v.at[pl.ds(0, EW)])

    for b in range(NSEG):
        cnt_s[b] = 0

    def hist_body(gi, carry):
        gb = pl.multiple_of(gi * 16, 16)
        rv = rowb_v[pl.ds(gb, 16)]
        cv = colb_v[pl.ds(gb, 16)]
        for l in range(16):
            plsc.addupdate_scatter(hist_v, [jnp.full((16,), rv[l], _i32)],
                                   ones16, mask=lane0)
            bb = ((cv[l] >> 7) * 205) >> 10
            cnt_s[bb] = cnt_s[bb] + 1
        return carry

    lax.fori_loop(0, EW // 16, hist_body, 0)

    off = 0
    for b in range(NSEG):
        cnt_s[16 + b] = off
        cnt_s[32 + b] = off
        pc = _div80(cnt_s[b] + (CH - 1)) * CH

        def pad_body(k, carry, _b=b):
            kv = jnp.full((16,), k, _i32)
            plsc.store_scatter(rout_v, [kv], jnp.zeros((16,), _i32), mask=lane0)
            plsc.store_scatter(cout_v, [kv],
                               jnp.full((16,), (_b + 1) * SEG, _i32),
                               mask=lane0)
            return carry

        lax.fori_loop(off + cnt_s[b], off + pc, pad_body, 0)
        off = off + pc

    def scat_body(gi, carry):
        gb = pl.multiple_of(gi * 16, 16)
        rv = rowb_v[pl.ds(gb, 16)]
        cv = colb_v[pl.ds(gb, 16)]
        for l in range(16):
            bb = ((cv[l] >> 7) * 205) >> 10
            p = cnt_s[32 + bb]
            pv = jnp.full((16,), p, _i32)
            plsc.store_scatter(rout_v, [pv], jnp.full((16,), rv[l], _i32),
                               mask=lane0)
            plsc.store_scatter(cout_v, [pv], jnp.full((16,), cv[l], _i32),
                               mask=lane0)
            cnt_s[32 + bb] = p + 1
        return carry

    lax.fori_loop(0, EW // 16, scat_body, 0)

    for b in range(NSEG):
        plsc.store_scatter(meta_v, [jnp.full((16,), b * 8, _i32)],
                           jnp.full((16,), cnt_s[16 + b], _i32), mask=lane0)
        plsc.store_scatter(meta_v, [jnp.full((16,), b * 8 + 1, _i32)],
                           jnp.full((16,), _div80(cnt_s[b] + (CH - 1)) * CH,
                                    _i32), mask=lane0)

    pltpu.sync_copy(rout_v, rows_out.at[pl.ds(w * CAP, CAP)])
    pltpu.sync_copy(cout_v, cols_out.at[pl.ds(w * CAP, CAP)])
    pltpu.sync_copy(meta_v, meta_out.at[pl.ds(w * 128, 128)])
    pltpu.sync_copy(hist_v.at[pl.ds(0, NPAD)], deg_out.at[pl.ds(w * NPAD, NPAD)])


@functools.partial(
    pl.kernel,
    out_type=jax.ShapeDtypeStruct((NW * SEG * D,), _f32),
    mesh=_mesh,
    compiler_params=pltpu.CompilerParams(needs_layout_passes=False),
    scratch_types=[
        pltpu.VMEM((CH,), _i32),
        pltpu.VMEM((CH,), _i32),
        pltpu.VMEM((CH,), _i32),
        pltpu.VMEM((CH,), _i32),
        pltpu.VMEM((CH, D), _f32),
        pltpu.VMEM((CH, D), _f32),
        pltpu.VMEM((AROWS * D,), _f32),
        pltpu.VMEM((NS * 128 + 16,), _i32),
        pltpu.SemaphoreType.DMA,
        pltpu.SemaphoreType.DMA,
        pltpu.SemaphoreType.DMA,
        pltpu.SemaphoreType.DMA,
    ],
)
def _sc_scatter(rows_hbm, cols_hbm, meta_hbm, g_hbm, zeros_hbm, out_hbm,
                ridxa_v, cidxa_v, ridxb_v, cidxb_v, msga_v, msgb_v,
                accum_v, meta_v, sia, sib, sga, sgb):
    c = lax.axis_index("c")
    b = lax.axis_index("s")
    pltpu.sync_copy(zeros_hbm, accum_v)
    pltpu.sync_copy(meta_hbm.at[pl.ds(c * NS * 128, NS * 128)],
                    meta_v.at[pl.ds(0, NS * 128)])
    nbase = b * SEG

    def accumulate(cidx_v, msg_v):
        for g16 in range(CH // 16):
            cv = cidx_v[pl.ds(g16 * 16, 16)]
            for l in range(16):
                e = g16 * 16 + l
                lb = pl.multiple_of((cv[l] - nbase) * D, 16)
                mvs = [msg_v[e, pl.ds(k * 16, 16)] for k in range(D // 16)]
                for k in range(D // 16):
                    plsc.addupdate(accum_v.at[pl.ds(lb + k * 16, 16)], mvs[k])

    def tile_body(t, carry):
        mv = meta_v[pl.ds(pl.multiple_of(t * 128 + b * 8, 8), 16)]
        off_t = mv[0]
        trip = _div80(mv[1])
        wbase = (c * NS + t) * CAP + off_t

        def chunk_off(j):
            return pl.multiple_of(wbase + j * CH, 8)

        @pl.when(trip > 0)
        def _():
            p = chunk_off(0)
            pltpu.sync_copy(rows_hbm.at[pl.ds(p, CH)], ridxa_v)
            pltpu.sync_copy(cols_hbm.at[pl.ds(p, CH)], cidxa_v)
            pltpu.make_async_copy(g_hbm.at[ridxa_v], msga_v, sga).start()

        @pl.when(trip > 1)
        def _():
            p = chunk_off(1)
            pltpu.make_async_copy(rows_hbm.at[pl.ds(p, CH)], ridxb_v, sib).start()
            pltpu.make_async_copy(cols_hbm.at[pl.ds(p, CH)], cidxb_v, sib).start()

        def pair_body(m, cy):
            j0 = 2 * m
            j1 = j0 + 1
            j2 = j0 + 2
            j3 = j0 + 3
            pltpu.make_async_copy(g_hbm.at[ridxa_v], msga_v, sga).wait()

            @pl.when(j1 < trip)
            def _():
                p = chunk_off(j1)
                pltpu.make_async_copy(rows_hbm.at[pl.ds(p, CH)], ridxb_v,
                                      sib).wait()
                pltpu.make_async_copy(cols_hbm.at[pl.ds(p, CH)], cidxb_v,
                                      sib).wait()
                pltpu.make_async_copy(g_hbm.at[ridxb_v], msgb_v, sgb).start()

            accumulate(cidxa_v, msga_v)

            @pl.when(j2 < trip)
            def _():
                p = chunk_off(j2)
                pltpu.make_async_copy(rows_hbm.at[pl.ds(p, CH)], ridxa_v,
                                      sia).start()
                pltpu.make_async_copy(cols_hbm.at[pl.ds(p, CH)], cidxa_v,
                                      sia).start()

            @pl.when(j1 < trip)
            def _():
                pltpu.make_async_copy(g_hbm.at[ridxb_v], msgb_v, sgb).wait()
                accumulate(cidxb_v, msgb_v)

                @pl.when(j3 < trip)
                def _():
                    p = chunk_off(j3)
                    pltpu.make_async_copy(rows_hbm.at[pl.ds(p, CH)], ridxb_v,
                                          sib).start()
                    pltpu.make_async_copy(cols_hbm.at[pl.ds(p, CH)], cidxb_v,
                                          sib).start()

            @pl.when(j2 < trip)
            def _():
                p = chunk_off(j2)
                pltpu.make_async_copy(rows_hbm.at[pl.ds(p, CH)], ridxa_v,
                                      sia).wait()
                pltpu.make_async_copy(cols_hbm.at[pl.ds(p, CH)], cidxa_v,
                                      sia).wait()
                pltpu.make_async_copy(g_hbm.at[ridxa_v], msga_v, sga).start()

            return cy

        lax.fori_loop(0, (trip + 1) >> 1, pair_body, 0)
        return carry

    lax.fori_loop(0, NS, tile_body, 0)

    pltpu.sync_copy(accum_v.at[pl.ds(0, SEG * D)],
                    out_hbm.at[pl.ds((c * NS + b) * SEG * D, SEG * D)])


NB = 5
BR = N // NB


def _prep_body(degp_ref, x_ref, dis_ref, g_ref):
    deg = jnp.sum(degp_ref[...], axis=1, keepdims=True)
    dis = jnp.where(deg > 0.0, lax.rsqrt(jnp.maximum(deg, 1e-12)), 0.0)
    dis_ref[...] = dis
    g_ref[...] = x_ref[...] * dis


_tc_prep = pl.pallas_call(
    _prep_body,
    grid=(NB,),
    in_specs=[
        pl.BlockSpec((BR, NW), lambda i: (i, 0)),
        pl.BlockSpec((BR, D), lambda i: (i, 0)),
    ],
    out_specs=[
        pl.BlockSpec((BR, 1), lambda i: (i, 0)),
        pl.BlockSpec((BR, D), lambda i: (i, 0)),
    ],
    out_shape=[
        jax.ShapeDtypeStruct((N, 1), _f32),
        jax.ShapeDtypeStruct((N, D), _f32),
    ],
)


def _layer_body(h_ref, t1_ref, dis_ref, w0_ref, w1_ref, b_ref, gam_ref, bet_ref,
                ho_ref, go_ref, acc_ref):
    p = pl.program_id(0)
    i = pl.program_id(1)
    dis = dis_ref[...]
    tx1 = -(dis * (t1_ref[0] + t1_ref[1]))
    u = (lax.dot_general(h_ref[...], w0_ref[...], (((1,), (1,)), ((), ())),
                         preferred_element_type=_f32)
         + lax.dot_general(tx1, w1_ref[...], (((1,), (1,)), ((), ())),
                           preferred_element_type=_f32)
         + b_ref[...])

    @pl.when(p == 0)
    def _():
        @pl.when(i == 0)
        def _():
            acc_ref[...] = jnp.zeros((8, D), _f32)
        acc_ref[0:1, :] += jnp.sum(u, axis=0, keepdims=True)

    @pl.when(p == 1)
    def _():
        @pl.when(i == 0)
        def _():
            acc_ref[2:3, :] = acc_ref[0:1, :] * (1.0 / N)
        d = u - acc_ref[2:3, :]
        acc_ref[1:2, :] += jnp.sum(d * d, axis=0, keepdims=True)

    @pl.when(p == 2)
    def _():
        @pl.when(i == 0)
        def _():
            acc_ref[3:4, :] = lax.rsqrt(acc_ref[1:2, :] * (1.0 / N) + EPS)
        hn = jnp.maximum((u - acc_ref[2:3, :]) * acc_ref[3:4, :]
                         * gam_ref[...] + bet_ref[...], 0.0)
        ho_ref[...] = hn
        go_ref[...] = hn * dis


_tc_layer = pl.pallas_call(
    _layer_body,
    grid=(3, NB),
    in_specs=[
        pl.BlockSpec((BR, D), lambda p, i: (i, 0)),
        pl.BlockSpec((NC, BR, D), lambda p, i: (0, i, 0)),
        pl.BlockSpec((BR, 1), lambda p, i: (i, 0)),
        pl.BlockSpec((D, D), lambda p, i: (0, 0)),
        pl.BlockSpec((D, D), lambda p, i: (0, 0)),
        pl.BlockSpec((1, D), lambda p, i: (0, 0)),
        pl.BlockSpec((1, D), lambda p, i: (0, 0)),
        pl.BlockSpec((1, D), lambda p, i: (0, 0)),
    ],
    out_specs=[
        pl.BlockSpec((BR, D), lambda p, i: (i, 0)),
        pl.BlockSpec((BR, D), lambda p, i: (i, 0)),
    ],
    out_shape=[
        jax.ShapeDtypeStruct((N, D), _f32),
        jax.ShapeDtypeStruct((N, D), _f32),
    ],
    scratch_shapes=[pltpu.VMEM((8, D), _f32)],
)


def _final_body(h_ref, bt_ref, fc1w_ref, fc1b_ref, bg_ref, bb_ref,
                fc2w_ref, fc2b_ref, out_ref, ps_ref, cnt_ref):
    i = pl.program_id(0)

    @pl.when(i == 0)
    def _():
        ps_ref[...] = jnp.zeros((104, D), _f32)
        cnt_ref[...] = jnp.zeros((104, 8), _f32)

    bt = bt_ref[...]
    M = (bt == lax.broadcasted_iota(_i32, (1, G), 1)).astype(_f32)
    ps_ref[0:G, :] += lax.dot_general(M, h_ref[...], (((0,), (0,)), ((), ())),
                                      preferred_element_type=_f32)
    cnt_ref[0:G, 0:1] += lax.dot_general(
        M, jnp.ones((BR, 1), _f32), (((0,), (0,)), ((), ())),
        preferred_element_type=_f32)

    @pl.when(i == NB - 1)
    def _():
        pooled = ps_ref[0:G, :] / jnp.maximum(cnt_ref[0:G, 0:1], 1.0)
        z = lax.dot_general(pooled, fc1w_ref[...], (((1,), (1,)), ((), ())),
                            preferred_element_type=_f32) + fc1b_ref[...]
        m = jnp.mean(z, axis=0, keepdims=True)
        v = jnp.mean((z - m) ** 2, axis=0, keepdims=True)
        z = jnp.maximum((z - m) * lax.rsqrt(v + EPS) * bg_ref[...] + bb_ref[...],
                        0.0)
        out_ref[...] = lax.dot_general(z, fc2w_ref[...], (((1,), (1,)), ((), ())),
                                       preferred_element_type=_f32) + fc2b_ref[...]


_tc_final = pl.pallas_call(
    _final_body,
    grid=(NB,),
    in_specs=[
        pl.BlockSpec((BR, D), lambda i: (i, 0)),
        pl.BlockSpec((BR, 1), lambda i: (i, 0)),
        pl.BlockSpec((H, D), lambda i: (0, 0)),
        pl.BlockSpec((1, H), lambda i: (0, 0)),
        pl.BlockSpec((1, H), lambda i: (0, 0)),
        pl.BlockSpec((1, H), lambda i: (0, 0)),
        pl.BlockSpec((O, H), lambda i: (0, 0)),
        pl.BlockSpec((1, O), lambda i: (0, 0)),
    ],
    out_specs=pl.BlockSpec((G, O), lambda i: (0, 0)),
    out_shape=jax.ShapeDtypeStruct((G, O), _f32),
    scratch_shapes=[pltpu.VMEM((104, D), _f32), pltpu.VMEM((104, 8), _f32)],
)


def kernel(x, edge_index, batch, cheb_w0, cheb_w1, cheb_b, bn_gamma, bn_beta,
           fc1_w, fc1_b, bnff_gamma, bnff_beta, fc2_w, fc2_b):
    row = edge_index[0]
    col = edge_index[1]
    zeros_i = jnp.zeros((NPAD,), _i32)
    zeros_a = jnp.zeros((AROWS * D,), _f32)

    rows_s, cols_s, meta, degh = _sc_bucket(row, col, zeros_i)
    degt = degh.reshape(NW, NPAD)[:, :N].astype(_f32).T
    dis, g = _tc_prep(degt, x)

    b2 = cheb_b.reshape(1, D)
    h = x
    for i in range(L):
        t1 = _sc_scatter(rows_s, cols_s, meta, g, zeros_a)
        t1 = t1.reshape(NC, NPAD, D)
        h, g = _tc_layer(h, t1, dis, cheb_w0, cheb_w1, b2,
                         bn_gamma[i].reshape(1, D), bn_beta[i].reshape(1, D))

    return _tc_final(h, batch.reshape(N, 1), fc1_w, fc1_b.reshape(1, H),
                     bnff_gamma.reshape(1, H), bnff_beta.reshape(1, H),
                     fc2_w, fc2_b.reshape(1, O))
